# speedup vs baseline: 22.3836x; 22.3836x over previous
import jax
import jax.numpy as jnp
from jax.experimental import pallas as pl
from jax.experimental.pallas import tpu as pltpu

_L = 256


def _body(x_ref, wb_ref, wc_ref, tri_ref, w1_ref, w2_ref, v1_ref, v2_ref,
          l1_ref, l2_ref, y_ref, h_ref):
    n2 = w1_ref.shape[1]
    n = n2 // 2
    t_idx = pl.program_id(1)

    @pl.when(t_idx == 0)
    def _():
        h_ref[...] = jnp.zeros_like(h_ref)

    xb = x_ref[0]
    z = jnp.dot(xb, wb_ref[...], preferred_element_type=jnp.float32)
    zs = jnp.concatenate([z[:, n:], z[:, :n]], axis=1)
    bp = w1_ref[...] * z + w2_ref[...] * zs
    c = jnp.dot(tri_ref[...], bp.astype(jnp.bfloat16),
                preferred_element_type=jnp.float32)
    h = h_ref[...]
    hs = jnp.concatenate([h[:, n:], h[:, :n]], axis=1)
    lh = l1_ref[...] * h + l2_ref[...] * hs
    cp = c + lh
    cps = jnp.concatenate([cp[:, n:], cp[:, :n]], axis=1)
    s = v1_ref[...] * cp + v2_ref[...] * cps
    h_ref[...] = s[_L - 1:_L, :]
    sx = jnp.concatenate([s.astype(jnp.bfloat16), xb], axis=1)
    y_ref[0] = jnp.dot(sx, wc_ref[...], preferred_element_type=jnp.float32)


def kernel(x, nu_log, theta_log, gamma_log, B_re, B_im, C_re, C_im, D):
    b_sz, t_len, d_in = x.shape
    d_out = D.shape[0]
    n = nu_log.shape[0]
    L = _L
    n_chunks = t_len // L

    nu = jnp.exp(nu_log)
    theta = jnp.exp(theta_log)
    gamma = jnp.exp(gamma_log)

    t = jnp.arange(L, dtype=jnp.float32)[:, None]
    ang = t * theta[None, :]
    ct, st = jnp.cos(ang), jnp.sin(ang)
    mag_pos = jnp.exp(-t * nu[None, :])
    mag_neg = jnp.exp(t * nu[None, :])
    v_re, v_im = mag_pos * ct, mag_pos * st
    w_re, w_im = mag_neg * ct, -(mag_neg * st)
    W1 = jnp.concatenate([w_re, w_re], axis=1)
    W2 = jnp.concatenate([-w_im, w_im], axis=1)
    V1 = jnp.concatenate([v_re, v_re], axis=1)
    V2 = jnp.concatenate([-v_im, v_im], axis=1)
    lam_re = jnp.exp(-nu) * jnp.cos(theta)
    lam_im = jnp.exp(-nu) * jnp.sin(theta)
    L1 = jnp.concatenate([lam_re, lam_re])[None, :]
    L2 = jnp.concatenate([-lam_im, lam_im])[None, :]

    Wb = jnp.concatenate([(B_re * gamma[:, None]).T,
                          (B_im * gamma[:, None]).T], axis=1).astype(jnp.bfloat16)
    Wc = jnp.concatenate([C_re.T, -C_im.T, D.T], axis=0).astype(jnp.bfloat16)
    tri = jnp.tril(jnp.ones((L, L), jnp.float32)).astype(jnp.bfloat16)
    xb = x.astype(jnp.bfloat16)

    const = lambda *_: (0, 0)
    grid = (b_sz, n_chunks)
    y = pl.pallas_call(
        _body,
        out_shape=jax.ShapeDtypeStruct((b_sz, t_len, d_out), jnp.float32),
        grid=grid,
        in_specs=[
            pl.BlockSpec((1, L, d_in), lambda b, tc: (b, tc, 0)),
            pl.BlockSpec((d_in, 2 * n), const),
            pl.BlockSpec((2 * n + d_in, d_out), const),
            pl.BlockSpec((L, L), const),
            pl.BlockSpec((L, 2 * n), const),
            pl.BlockSpec((L, 2 * n), const),
            pl.BlockSpec((L, 2 * n), const),
            pl.BlockSpec((L, 2 * n), const),
            pl.BlockSpec((1, 2 * n), const),
            pl.BlockSpec((1, 2 * n), const),
        ],
        out_specs=pl.BlockSpec((1, L, d_out), lambda b, tc: (b, tc, 0)),
        scratch_shapes=[pltpu.VMEM((1, 2 * n), jnp.float32)],
        compiler_params=pltpu.CompilerParams(
            dimension_semantics=("parallel", "arbitrary"),
            vmem_limit_bytes=56 * 1024 * 1024,
        ),
        name="lru_fused",
    )(xb, Wb, Wc, tri, W1, W2, V1, V2, L1, L2)
    return y


# speedup vs baseline: 22.6059x; 1.0099x over previous
import jax
import jax.numpy as jnp
from jax.experimental import pallas as pl
from jax.experimental.pallas import tpu as pltpu

_L = 256


def _body(x_ref, wb_ref, wc_ref, tri_ref, wr_ref, wi_ref, vr_ref, vi_ref,
          lam_ref, y_ref, h_ref):
    n = wr_ref.shape[1]
    t_idx = pl.program_id(1)

    @pl.when(t_idx == 0)
    def _():
        h_ref[...] = jnp.zeros_like(h_ref)

    xb = x_ref[0]
    z = jnp.dot(xb, wb_ref[...], preferred_element_type=jnp.float32)
    zr, zi = z[:, :n], z[:, n:]
    wr, wi = wr_ref[...], wi_ref[...]
    bp = jnp.concatenate([wr * zr - wi * zi, wi * zr + wr * zi], axis=1)
    c = jnp.dot(tri_ref[...], bp.astype(jnp.bfloat16),
                preferred_element_type=jnp.float32)
    h = h_ref[...]
    hr, hi = h[:, :n], h[:, n:]
    lr, li = lam_ref[...][:, :n], lam_ref[...][:, n:]
    cr = c[:, :n] + (lr * hr - li * hi)
    ci = c[:, n:] + (li * hr + lr * hi)
    vr, vi = vr_ref[...], vi_ref[...]
    sr = vr * cr - vi * ci
    si = vi * cr + vr * ci
    h_ref[...] = jnp.concatenate([sr[_L - 1:_L, :], si[_L - 1:_L, :]], axis=1)
    sx = jnp.concatenate([sr.astype(jnp.bfloat16), si.astype(jnp.bfloat16),
                          xb], axis=1)
    y_ref[0] = jnp.dot(sx, wc_ref[...], preferred_element_type=jnp.float32)


def kernel(x, nu_log, theta_log, gamma_log, B_re, B_im, C_re, C_im, D):
    b_sz, t_len, d_in = x.shape
    d_out = D.shape[0]
    n = nu_log.shape[0]
    L = _L
    n_chunks = t_len // L

    nu = jnp.exp(nu_log)
    theta = jnp.exp(theta_log)
    gamma = jnp.exp(gamma_log)

    t = jnp.arange(L, dtype=jnp.float32)[:, None]
    ang = t * theta[None, :]
    ct, st = jnp.cos(ang), jnp.sin(ang)
    mag_pos = jnp.exp(-t * nu[None, :])
    mag_neg = jnp.exp(t * nu[None, :])
    Vr, Vi = mag_pos * ct, mag_pos * st
    Wr, Wi = mag_neg * ct, -(mag_neg * st)
    lam_re = jnp.exp(-nu) * jnp.cos(theta)
    lam_im = jnp.exp(-nu) * jnp.sin(theta)
    Lam = jnp.concatenate([lam_re, lam_im])[None, :]

    Wb = jnp.concatenate([(B_re * gamma[:, None]).T,
                          (B_im * gamma[:, None]).T], axis=1).astype(jnp.bfloat16)
    Wc = jnp.concatenate([C_re.T, -C_im.T, D.T], axis=0).astype(jnp.bfloat16)
    tri = jnp.tril(jnp.ones((L, L), jnp.float32)).astype(jnp.bfloat16)
    xb = x.astype(jnp.bfloat16)

    const = lambda *_: (0, 0)
    grid = (b_sz, n_chunks)
    y = pl.pallas_call(
        _body,
        out_shape=jax.ShapeDtypeStruct((b_sz, t_len, d_out), jnp.float32),
        grid=grid,
        in_specs=[
            pl.BlockSpec((1, L, d_in), lambda b, tc: (b, tc, 0)),
            pl.BlockSpec((d_in, 2 * n), const),
            pl.BlockSpec((2 * n + d_in, d_out), const),
            pl.BlockSpec((L, L), const),
            pl.BlockSpec((L, n), const),
            pl.BlockSpec((L, n), const),
            pl.BlockSpec((L, n), const),
            pl.BlockSpec((L, n), const),
            pl.BlockSpec((1, 2 * n), const),
        ],
        out_specs=pl.BlockSpec((1, L, d_out), lambda b, tc: (b, tc, 0)),
        scratch_shapes=[pltpu.VMEM((1, 2 * n), jnp.float32)],
        compiler_params=pltpu.CompilerParams(
            dimension_semantics=("parallel", "arbitrary"),
            vmem_limit_bytes=56 * 1024 * 1024,
        ),
        name="lru_fused",
    )(xb, Wb, Wc, tri, Wr, Wi, Vr, Vi, Lam)
    return y


# speedup vs baseline: 24.4968x; 1.0836x over previous
import jax
import jax.numpy as jnp
from jax.experimental import pallas as pl
from jax.experimental.pallas import tpu as pltpu

_L = 512


def _body(x_ref, wb_ref, wc_ref, tri_ref, wr_ref, wi_ref, vr_ref, vi_ref,
          lam_ref, y_ref, h_ref):
    n = wr_ref.shape[1]
    t_idx = pl.program_id(1)

    @pl.when(t_idx == 0)
    def _():
        h_ref[...] = jnp.zeros_like(h_ref)

    xb = x_ref[0]
    z = jnp.dot(xb, wb_ref[...], preferred_element_type=jnp.float32)
    zr, zi = z[:, :n], z[:, n:]
    wr, wi = wr_ref[...], wi_ref[...]
    bp = jnp.concatenate([wr * zr - wi * zi, wi * zr + wr * zi], axis=1)
    c = jnp.dot(tri_ref[...], bp.astype(jnp.bfloat16),
                preferred_element_type=jnp.float32)
    h = h_ref[...]
    hr, hi = h[:, :n], h[:, n:]
    lr, li = lam_ref[...][:, :n], lam_ref[...][:, n:]
    cr = c[:, :n] + (lr * hr - li * hi)
    ci = c[:, n:] + (li * hr + lr * hi)
    vr, vi = vr_ref[...], vi_ref[...]
    sr = vr * cr - vi * ci
    si = vi * cr + vr * ci
    h_ref[...] = jnp.concatenate([sr[_L - 1:_L, :], si[_L - 1:_L, :]], axis=1)
    sx = jnp.concatenate([sr.astype(jnp.bfloat16), si.astype(jnp.bfloat16),
                          xb], axis=1)
    y_ref[0] = jnp.dot(sx, wc_ref[...], preferred_element_type=jnp.float32)


def kernel(x, nu_log, theta_log, gamma_log, B_re, B_im, C_re, C_im, D):
    b_sz, t_len, d_in = x.shape
    d_out = D.shape[0]
    n = nu_log.shape[0]
    L = _L
    n_chunks = t_len // L

    nu = jnp.exp(nu_log)
    theta = jnp.exp(theta_log)
    gamma = jnp.exp(gamma_log)

    t = jnp.arange(L, dtype=jnp.float32)[:, None]
    ang = t * theta[None, :]
    ct, st = jnp.cos(ang), jnp.sin(ang)
    mag_pos = jnp.exp(-t * nu[None, :])
    mag_neg = jnp.exp(t * nu[None, :])
    Vr, Vi = mag_pos * ct, mag_pos * st
    Wr, Wi = mag_neg * ct, -(mag_neg * st)
    lam_re = jnp.exp(-nu) * jnp.cos(theta)
    lam_im = jnp.exp(-nu) * jnp.sin(theta)
    Lam = jnp.concatenate([lam_re, lam_im])[None, :]

    Wb = jnp.concatenate([(B_re * gamma[:, None]).T,
                          (B_im * gamma[:, None]).T], axis=1).astype(jnp.bfloat16)
    Wc = jnp.concatenate([C_re.T, -C_im.T, D.T], axis=0).astype(jnp.bfloat16)
    tri = jnp.tril(jnp.ones((L, L), jnp.float32)).astype(jnp.bfloat16)
    xb = x.astype(jnp.bfloat16)

    const = lambda *_: (0, 0)
    grid = (b_sz, n_chunks)
    y = pl.pallas_call(
        _body,
        out_shape=jax.ShapeDtypeStruct((b_sz, t_len, d_out), jnp.float32),
        grid=grid,
        in_specs=[
            pl.BlockSpec((1, L, d_in), lambda b, tc: (b, tc, 0)),
            pl.BlockSpec((d_in, 2 * n), const),
            pl.BlockSpec((2 * n + d_in, d_out), const),
            pl.BlockSpec((L, L), const),
            pl.BlockSpec((L, n), const),
            pl.BlockSpec((L, n), const),
            pl.BlockSpec((L, n), const),
            pl.BlockSpec((L, n), const),
            pl.BlockSpec((1, 2 * n), const),
        ],
        out_specs=pl.BlockSpec((1, L, d_out), lambda b, tc: (b, tc, 0)),
        scratch_shapes=[pltpu.VMEM((1, 2 * n), jnp.float32)],
        compiler_params=pltpu.CompilerParams(
            dimension_semantics=("parallel", "arbitrary"),
            vmem_limit_bytes=56 * 1024 * 1024,
        ),
        name="lru_fused",
    )(xb, Wb, Wc, tri, Wr, Wi, Vr, Vi, Lam)
    return y
